# Initial kernel scaffold; baseline (speedup 1.0000x reference)
import jax
import jax.numpy as jnp
from jax.experimental import pallas as pl


def kernel(x, relationsedge_indices_relations, edge_type, W_rel1, W_root1, b1, W_rel2, W_root2, b2):
    raise NotImplementedError("write your pallas kernel here")


# speedup vs baseline: 4.0088x; 4.0088x over previous
import functools

import jax
import jax.numpy as jnp
from jax import lax
from jax.experimental import pallas as pl
from jax.experimental.pallas import tpu as pltpu
from jax.experimental.pallas import tpu_sc as plsc

N, E, D, H, O, R = 10000, 320000, 128, 128, 128, 2

NC, NS, LANES = 2, 16, 16
CH = 128
TRASH = N
N_PAD = 10112
ROWS_PER_TILE = N_PAD // NS
E_PAD = 323584
CHUNKS_A = E_PAD // (NS * CH)
CHUNKS_B = E_PAD // (NC * NS * CH)
NCH = N_PAD // CH
KMAX = (NCH + NS - 1) // NS

_mesh = plsc.VectorSubcoreMesh(core_axis_name="c", subcore_axis_name="s")


@functools.partial(
    pl.kernel,
    out_type=[
        jax.ShapeDtypeStruct((NC, N_PAD, D), jnp.float32),
        jax.ShapeDtypeStruct((NC, N_PAD, LANES), jnp.float32),
    ],
    mesh=_mesh,
    compiler_params=pltpu.CompilerParams(use_tc_tiling_on_sc=False),
    scratch_types=[
        pltpu.VMEM((CH, D), jnp.float32),
        pltpu.VMEM((1, CH), jnp.int32),
        pltpu.VMEM((1, CH), jnp.int32),
        pltpu.VMEM((CH, LANES), jnp.float32),
        pltpu.VMEM((CH, LANES), jnp.float32),
        pltpu.VMEM_SHARED((N_PAD, D), jnp.float32),
        pltpu.VMEM_SHARED((N_PAD, LANES), jnp.float32),
    ],
)
def _sc_pass_a(x_hbm, src_hbm, dstm_hbm, z128_hbm, z16_hbm, ones_hbm,
               agg_out, cnt_out,
               rows_v, sidx_v, didx_v, ones_v, cbuf_v, agg_sh, cnt_sh):
    c = lax.axis_index("c")
    s = lax.axis_index("s")

    pltpu.sync_copy(z128_hbm.at[pl.ds(0, CH)], rows_v)
    pltpu.sync_copy(z16_hbm.at[pl.ds(0, CH)], cbuf_v)
    for k in range(KMAX):
        ch = s + NS * k

        @pl.when(ch < NCH)
        def _():
            r = ch * CH
            pltpu.sync_copy(rows_v, agg_sh.at[pl.ds(r, CH)])
            pltpu.sync_copy(cbuf_v, cnt_sh.at[pl.ds(r, CH)])

    pltpu.sync_copy(ones_hbm, ones_v)
    plsc.subcore_barrier()

    def body(j, carry):
        base = (s * CHUNKS_A + j) * CH
        pltpu.sync_copy(src_hbm.at[pl.ds(base, CH)], sidx_v.at[0])
        pltpu.sync_copy(dstm_hbm.at[pl.ds(c * E_PAD + base, CH)], didx_v.at[0])
        pltpu.sync_copy(x_hbm.at[sidx_v.at[0]], rows_v)
        pltpu.sync_copy(rows_v, agg_sh.at[didx_v.at[0]], add=True)
        pltpu.sync_copy(ones_v, cnt_sh.at[didx_v.at[0]], add=True)
        return carry

    lax.fori_loop(0, CHUNKS_A, body, 0)
    plsc.subcore_barrier()

    for k in range(KMAX):
        ch = s + NS * k

        @pl.when(ch < NCH)
        def _():
            r = ch * CH
            pltpu.sync_copy(agg_sh.at[pl.ds(r, CH)], rows_v)
            pltpu.sync_copy(rows_v, agg_out.at[c, pl.ds(r, CH)])
            pltpu.sync_copy(cnt_sh.at[pl.ds(r, CH)], cbuf_v)
            pltpu.sync_copy(cbuf_v, cnt_out.at[c, pl.ds(r, CH)])


@functools.partial(
    pl.kernel,
    out_type=jax.ShapeDtypeStruct((NC, N_PAD, D), jnp.float32),
    mesh=_mesh,
    compiler_params=pltpu.CompilerParams(use_tc_tiling_on_sc=False),
    scratch_types=[
        pltpu.VMEM((CH, D), jnp.float32),
        pltpu.VMEM((1, CH), jnp.int32),
        pltpu.VMEM((1, CH), jnp.int32),
        pltpu.VMEM_SHARED((N_PAD, D), jnp.float32),
    ],
)
def _sc_pass_b(h_hbm, src_hbm, dstp_hbm, z128_hbm,
               part_out, rows_v, sidx_v, didx_v, agg_sh):
    c = lax.axis_index("c")
    s = lax.axis_index("s")

    pltpu.sync_copy(z128_hbm.at[pl.ds(0, CH)], rows_v)
    for k in range(KMAX):
        ch = s + NS * k

        @pl.when(ch < NCH)
        def _():
            pltpu.sync_copy(rows_v, agg_sh.at[pl.ds(ch * CH, CH)])

    plsc.subcore_barrier()

    def body(j, carry):
        base = ((c * NS + s) * CHUNKS_B + j) * CH
        pltpu.sync_copy(src_hbm.at[pl.ds(base, CH)], sidx_v.at[0])
        pltpu.sync_copy(dstp_hbm.at[pl.ds(base, CH)], didx_v.at[0])
        pltpu.sync_copy(h_hbm.at[sidx_v.at[0]], rows_v)
        pltpu.sync_copy(rows_v, agg_sh.at[didx_v.at[0]], add=True)
        return carry

    lax.fori_loop(0, CHUNKS_B, body, 0)
    plsc.subcore_barrier()

    for k in range(KMAX):
        ch = s + NS * k

        @pl.when(ch < NCH)
        def _():
            r = ch * CH
            pltpu.sync_copy(agg_sh.at[pl.ds(r, CH)], rows_v)
            pltpu.sync_copy(rows_v, part_out.at[c, pl.ds(r, CH)])


_BLK = 400


def _tc1_body(x_ref, a0_ref, a1_ref, c0_ref, c1_ref,
              wr_ref, w0_ref, w1_ref, b_ref, h_ref):
    cnt0 = jnp.maximum(c0_ref[:, 0:1], 1.0)
    cnt1 = jnp.maximum(c1_ref[:, 0:1], 1.0)
    acc = jnp.dot(x_ref[...], wr_ref[...], preferred_element_type=jnp.float32)
    acc += jnp.dot(a0_ref[...] / cnt0, w0_ref[...],
                   preferred_element_type=jnp.float32)
    acc += jnp.dot(a1_ref[...] / cnt1, w1_ref[...],
                   preferred_element_type=jnp.float32)
    h_ref[...] = jnp.maximum(acc + b_ref[...], 0.0)


def _tc2_body(p0_ref, p1_ref, h_ref, wrel_ref, wroot_ref, b_ref, o_ref):
    acc = jnp.dot(p0_ref[...] + p1_ref[...], wrel_ref[...],
                  preferred_element_type=jnp.float32)
    acc += jnp.dot(h_ref[...], wroot_ref[...],
                   preferred_element_type=jnp.float32)
    o_ref[...] = acc + b_ref[...]


def _row_blk(i):
    return (i, 0)


def _whole(i):
    return (0, 0)


def kernel(x, relationsedge_indices_relations, edge_type, W_rel1, W_root1, b1,
           W_rel2, W_root2, b2):
    edge_index = relationsedge_indices_relations[-1]
    src = edge_index[0].astype(jnp.int32)
    dst = edge_index[1].astype(jnp.int32)
    et = edge_type.astype(jnp.int32)

    pad = E_PAD - E
    src_p = jnp.concatenate([src, jnp.zeros((pad,), jnp.int32)])
    dst_p = jnp.concatenate([dst, jnp.full((pad,), TRASH, jnp.int32)])
    et_p = jnp.concatenate([et, jnp.full((pad,), R, jnp.int32)])
    dstm = jnp.concatenate(
        [jnp.where(et_p == r, dst_p, TRASH) for r in range(R)])

    z128 = jnp.zeros((N_PAD, D), jnp.float32)
    z16 = jnp.zeros((N_PAD, LANES), jnp.float32)
    ones = jnp.ones((CH, LANES), jnp.float32)

    agg, cnt = _sc_pass_a(x, src_p, dstm, z128, z16, ones)

    rowspec = pl.BlockSpec((_BLK, D), _row_blk)
    cntspec = pl.BlockSpec((_BLK, LANES), _row_blk)
    wspec = pl.BlockSpec((D, D), _whole)
    bspec = pl.BlockSpec((1, D), _whole)

    h = pl.pallas_call(
        _tc1_body,
        grid=(N // _BLK,),
        in_specs=[rowspec, rowspec, rowspec, cntspec, cntspec,
                  wspec, wspec, wspec, bspec],
        out_specs=rowspec,
        out_shape=jax.ShapeDtypeStruct((N, D), jnp.float32),
    )(x, agg[0, :N], agg[1, :N], cnt[0, :N], cnt[1, :N],
      W_root1, W_rel1[0], W_rel1[1], b1.reshape(1, D))

    part = _sc_pass_b(h, src_p, dst_p, z128)

    out2 = pl.pallas_call(
        _tc2_body,
        grid=(N // _BLK,),
        in_specs=[rowspec, rowspec, rowspec, wspec, wspec, bspec],
        out_specs=rowspec,
        out_shape=jax.ShapeDtypeStruct((N, O), jnp.float32),
    )(part[0, :N], part[1, :N], h, W_rel2, W_root2, b2.reshape(1, O))

    return out2.reshape(-1, 1, O)
